# baseline (speedup 1.0000x reference)
import functools

import jax
import jax.numpy as jnp
from jax import lax
from jax.experimental import pallas as pl
from jax.experimental.pallas import tpu as pltpu
from jax.experimental.pallas import tpu_sc as plsc

_B = 4096
_L = 50
_EMB = 128
_NC = 2
_NS = 16
_NW = _NC * _NS
_CB = _B // _NW
_CH = 128
_NCHUNK = _L * (_CB // _CH)
_NSLOT = 2
_LANES = 16


def _emb_body(words_hbm, prefixes_hbm, suffixes_hbm,
              ww_hbm, wp_hbm, ws_hbm, out_hbm,
              widx, pidx, sidx,
              accw0, accw1,
              bufp0, bufp1,
              bufs0, bufs1,
              semw0, semw1,
              semp0, semp1,
              sems0, sems1,
              semo0, semo1):
    wid = lax.axis_index("s") * _NC + lax.axis_index("c")
    b0 = wid * _CB

    accw = (accw0, accw1)
    bufp = (bufp0, bufp1)
    bufs = (bufs0, bufs1)
    semw = (semw0, semw1)
    semp = (semp0, semp1)
    sems = (sems0, sems1)
    semo = (semo0, semo1)

    pltpu.sync_copy(words_hbm.at[:, pl.ds(b0, _CB)], widx)
    pltpu.sync_copy(prefixes_hbm.at[:, pl.ds(b0, _CB)], pidx)
    pltpu.sync_copy(suffixes_hbm.at[:, pl.ds(b0, _CB)], sidx)

    def idx_slice(idx, c):
        return idx.at[c, pl.ds(0, _CH)]

    def out_slice(c):
        return out_hbm.at[c, pl.ds(b0, _CH), :]

    def start_gathers(c, slot):
        pltpu.async_copy(ww_hbm.at[idx_slice(widx, c)], accw[slot],
                         semw[slot])
        pltpu.async_copy(wp_hbm.at[idx_slice(pidx, c)], bufp[slot],
                         semp[slot])
        pltpu.async_copy(ws_hbm.at[idx_slice(sidx, c)], bufs[slot],
                         sems[slot])

    def wait_gathers(c, slot):
        pltpu.make_async_copy(ww_hbm.at[idx_slice(widx, c)], accw[slot],
                              semw[slot]).wait()
        pltpu.make_async_copy(wp_hbm.at[idx_slice(pidx, c)], bufp[slot],
                              semp[slot]).wait()
        pltpu.make_async_copy(ws_hbm.at[idx_slice(sidx, c)], bufs[slot],
                              sems[slot]).wait()

    def start_store(c, slot):
        pltpu.async_copy(accw[slot], out_slice(c), semo[slot])

    def wait_store(c, slot):
        pltpu.make_async_copy(accw[slot], out_slice(c), semo[slot]).wait()

    for c in range(_NSLOT - 1):
        start_gathers(c, c)

    def quad_body(c4, carry):
        for b in range(_NSLOT):
            c = c4 * _NSLOT + b
            nslot = (b + _NSLOT - 1) % _NSLOT
            wait_gathers(c, b)

            @pl.when(c >= 1)
            def _():
                wait_store(c - 1, nslot)

            @pl.when(c + _NSLOT - 1 < _NCHUNK)
            def _():
                start_gathers(c + _NSLOT - 1, nslot)

            acc = accw[b]
            bp = bufp[b]
            bs = bufs[b]

            def row_body(r, rc, acc=acc, bp=bp, bs=bs):
                for j in range(_EMB // _LANES):
                    sl = pl.ds(j * _LANES, _LANES)
                    plsc.addupdate(acc.at[r, sl], bp[r, sl] + bs[r, sl])
                return rc

            lax.fori_loop(0, _CH, row_body, 0, unroll=2)

            start_store(c, b)
        return carry

    lax.fori_loop(0, _NCHUNK // _NSLOT, quad_body, 0, unroll=False)
    wait_store(_NCHUNK - 1, (_NCHUNK - 1) % _NSLOT)


@functools.partial(jax.jit, static_argnums=())
def _emb_call(words_t, prefixes_t, suffixes_t, ww, wp, ws):
    mesh = plsc.VectorSubcoreMesh(core_axis_name="c", subcore_axis_name="s")
    fn = pl.kernel(
        _emb_body,
        out_type=jax.ShapeDtypeStruct((_L, _B, _EMB), jnp.float32),
        mesh=mesh,
        scratch_types=(
            [pltpu.VMEM((_L, _CB), jnp.int32)] * 3
            + [pltpu.VMEM((_CH, _EMB), jnp.float32)] * (3 * _NSLOT)
            + [pltpu.SemaphoreType.DMA] * (4 * _NSLOT)
        ),
    )
    return fn(words_t, prefixes_t, suffixes_t, ww, wp, ws)


def kernel(words, prefixes, suffixes, W_word, W_prefix, W_suffix):
    out_t = _emb_call(words.T, prefixes.T, suffixes.T,
                      W_word, W_prefix, W_suffix)
    return out_t.transpose(1, 0, 2)


# speedup vs baseline: 1.0604x; 1.0604x over previous
import functools

import jax
import jax.numpy as jnp
from jax import lax
from jax.experimental import pallas as pl
from jax.experimental.pallas import tpu as pltpu
from jax.experimental.pallas import tpu_sc as plsc

_B = 4096
_L = 50
_EMB = 128
_NC = 2
_NS = 16
_NW = _NC * _NS
_CB = _B // _NW
_CH = 64
_NCHUNK = _L * (_CB // _CH)
_NSLOT = 4
_LANES = 16


def _emb_body(words_hbm, prefixes_hbm, suffixes_hbm,
              ww_hbm, wp_hbm, ws_hbm, out_hbm,
              widx, pidx, sidx,
              accw0, accw1, accw2, accw3,
              bufp0, bufp1, bufp2, bufp3,
              bufs0, bufs1, bufs2, bufs3,
              semw0, semw1, semw2, semw3,
              semp0, semp1, semp2, semp3,
              sems0, sems1, sems2, sems3,
              semo0, semo1, semo2, semo3):
    wid = lax.axis_index("s") * _NC + lax.axis_index("c")
    b0 = wid * _CB

    accw = (accw0, accw1, accw2, accw3)
    bufp = (bufp0, bufp1, bufp2, bufp3)
    bufs = (bufs0, bufs1, bufs2, bufs3)
    semw = (semw0, semw1, semw2, semw3)
    semp = (semp0, semp1, semp2, semp3)
    sems = (sems0, sems1, sems2, sems3)
    semo = (semo0, semo1, semo2, semo3)

    pltpu.sync_copy(words_hbm.at[:, pl.ds(b0, _CB)], widx)
    pltpu.sync_copy(prefixes_hbm.at[:, pl.ds(b0, _CB)], pidx)
    pltpu.sync_copy(suffixes_hbm.at[:, pl.ds(b0, _CB)], sidx)

    def idx_slice(idx, c):
        return idx.at[c // 2, pl.ds((c % 2) * _CH, _CH)]

    def out_slice(c):
        return out_hbm.at[c // 2, pl.ds(b0 + (c % 2) * _CH, _CH), :]

    def start_gathers(c, slot):
        pltpu.async_copy(ww_hbm.at[idx_slice(widx, c)], accw[slot],
                         semw[slot])
        pltpu.async_copy(wp_hbm.at[idx_slice(pidx, c)], bufp[slot],
                         semp[slot])
        pltpu.async_copy(ws_hbm.at[idx_slice(sidx, c)], bufs[slot],
                         sems[slot])

    def wait_gathers(c, slot):
        pltpu.make_async_copy(ww_hbm.at[idx_slice(widx, c)], accw[slot],
                              semw[slot]).wait()
        pltpu.make_async_copy(wp_hbm.at[idx_slice(pidx, c)], bufp[slot],
                              semp[slot]).wait()
        pltpu.make_async_copy(ws_hbm.at[idx_slice(sidx, c)], bufs[slot],
                              sems[slot]).wait()

    def start_store(c, slot):
        pltpu.async_copy(accw[slot], out_slice(c), semo[slot])

    def wait_store(c, slot):
        pltpu.make_async_copy(accw[slot], out_slice(c), semo[slot]).wait()

    for c in range(_NSLOT - 1):
        start_gathers(c, c)

    def quad_body(c4, carry):
        for b in range(_NSLOT):
            c = c4 * _NSLOT + b
            nslot = (b + _NSLOT - 1) % _NSLOT
            wait_gathers(c, b)

            @pl.when(c >= 1)
            def _():
                wait_store(c - 1, nslot)

            @pl.when(c + _NSLOT - 1 < _NCHUNK)
            def _():
                start_gathers(c + _NSLOT - 1, nslot)

            acc = accw[b]
            bp = bufp[b]
            bs = bufs[b]

            def row_body(r, rc, acc=acc, bp=bp, bs=bs):
                for j in range(_EMB // _LANES):
                    sl = pl.ds(j * _LANES, _LANES)
                    plsc.addupdate(acc.at[r, sl], bp[r, sl] + bs[r, sl])
                return rc

            lax.fori_loop(0, _CH, row_body, 0, unroll=2)

            start_store(c, b)
        return carry

    lax.fori_loop(0, _NCHUNK // _NSLOT, quad_body, 0, unroll=False)
    wait_store(_NCHUNK - 1, (_NCHUNK - 1) % _NSLOT)


@functools.partial(jax.jit, static_argnums=())
def _emb_call(words_t, prefixes_t, suffixes_t, ww, wp, ws):
    mesh = plsc.VectorSubcoreMesh(core_axis_name="c", subcore_axis_name="s")
    fn = pl.kernel(
        _emb_body,
        out_type=jax.ShapeDtypeStruct((_L, _B, _EMB), jnp.float32),
        mesh=mesh,
        scratch_types=(
            [pltpu.VMEM((_L, _CB), jnp.int32)] * 3
            + [pltpu.VMEM((_CH, _EMB), jnp.float32)] * (3 * _NSLOT)
            + [pltpu.SemaphoreType.DMA] * (4 * _NSLOT)
        ),
    )
    return fn(words_t, prefixes_t, suffixes_t, ww, wp, ws)


def kernel(words, prefixes, suffixes, W_word, W_prefix, W_suffix):
    out_t = _emb_call(words.T, prefixes.T, suffixes.T,
                      W_word, W_prefix, W_suffix)
    return out_t.transpose(1, 0, 2)
